# baseline (speedup 1.0000x reference)
import jax
import jax.numpy as jnp
from jax import lax
from jax.experimental import pallas as pl
from jax.experimental.pallas import tpu as pltpu
from jax.experimental.pallas import tpu_sc as plsc

NC, NS = 2, 16
NW = NC * NS
CP = 32


def _sc_body(tok, pos, out, pos_buf, tb):
    B = tok.shape[0]
    SE = tok.shape[1]
    E = pos_buf.shape[0] // CP
    spw = SE // E // NW
    wid = lax.axis_index("s") * NC + lax.axis_index("c")
    base = wid * spw
    nvec = CP * E // 16
    for chunk in range(spw // CP):
        s0 = (base + chunk * CP) * E
        pltpu.sync_copy(pos.at[pl.ds(s0, CP * E)], pos_buf)
        for b in range(B):
            pltpu.sync_copy(tok.at[b, pl.ds(s0, CP * E)], tb)

            def body(i, carry):
                sl = pl.ds(i * 16, 16)
                plsc.addupdate(tb.at[sl], pos_buf[sl])
                return carry

            lax.fori_loop(0, nvec, body, 0, unroll=8)
            pltpu.sync_copy(tb, out.at[b, pl.ds(s0, CP * E)])


def kernel(token_embedding, pos_table):
    B, S, E = token_embedding.shape
    tok2 = token_embedding.reshape(B, S * E)
    pos2 = pos_table.reshape(pos_table.shape[0] * E)
    f = pl.kernel(
        _sc_body,
        out_type=jax.ShapeDtypeStruct((B, S * E), token_embedding.dtype),
        mesh=plsc.VectorSubcoreMesh(core_axis_name="c", subcore_axis_name="s"),
        scratch_types=[
            pltpu.VMEM((CP * E,), jnp.float32),
            pltpu.VMEM((CP * E,), jnp.float32),
        ],
    )
    return f(tok2, pos2).reshape(B, S, E)


# speedup vs baseline: 1.0360x; 1.0360x over previous
import jax
import jax.numpy as jnp
from jax import lax
from jax.experimental import pallas as pl
from jax.experimental.pallas import tpu as pltpu
from jax.experimental.pallas import tpu_sc as plsc

NC, NS = 2, 16
NW = NC * NS
CP = 32


def _sc_body(tok, pos, out, pos_buf, tb0, tb1, ss0, ss1):
    B = tok.shape[0]
    SE = tok.shape[1]
    E = pos_buf.shape[0] // CP
    spw = SE // E // NW
    wid = lax.axis_index("s") * NC + lax.axis_index("c")
    base = wid * spw
    nvec = CP * E // 16
    tbufs = (tb0, tb1)
    ssems = (ss0, ss1)
    store_d = [None, None]
    u = 0
    for chunk in range(spw // CP):
        s0 = (base + chunk * CP) * E
        pltpu.sync_copy(pos.at[pl.ds(s0, CP * E)], pos_buf)
        for b in range(B):
            cur = u % 2
            if store_d[cur] is not None:
                store_d[cur].wait()
            tb = tbufs[cur]
            pltpu.sync_copy(tok.at[b, pl.ds(s0, CP * E)], tb)

            def body(i, carry):
                sl = pl.ds(i * 16, 16)
                plsc.addupdate(tb.at[sl], pos_buf[sl])
                return carry

            lax.fori_loop(0, nvec, body, 0, unroll=8)
            store_d[cur] = pltpu.async_copy(
                tb, out.at[b, pl.ds(s0, CP * E)], ssems[cur])
            u += 1
    store_d[0].wait()
    store_d[1].wait()


def kernel(token_embedding, pos_table):
    B, S, E = token_embedding.shape
    tok2 = token_embedding.reshape(B, S * E)
    pos2 = pos_table.reshape(pos_table.shape[0] * E)
    f = pl.kernel(
        _sc_body,
        out_type=jax.ShapeDtypeStruct((B, S * E), token_embedding.dtype),
        mesh=plsc.VectorSubcoreMesh(core_axis_name="c", subcore_axis_name="s"),
        scratch_types=[
            pltpu.VMEM((CP * E,), jnp.float32),
            pltpu.VMEM((CP * E,), jnp.float32),
            pltpu.VMEM((CP * E,), jnp.float32),
            pltpu.SemaphoreType.DMA,
            pltpu.SemaphoreType.DMA,
        ],
    )
    return f(tok2, pos2).reshape(B, S, E)


# speedup vs baseline: 1.2924x; 1.2476x over previous
import jax
import jax.numpy as jnp
from jax import lax
from jax.experimental import pallas as pl
from jax.experimental.pallas import tpu as pltpu
from jax.experimental.pallas import tpu_sc as plsc

NC, NS = 2, 16
NW = NC * NS
CP = 32


def _sc_body(tok, pos, out, pos_buf, tb0, tb1, ss0, ss1):
    B, S, E = tok.shape
    spw = S // NW
    wid = lax.axis_index("s") * NC + lax.axis_index("c")
    base = wid * spw
    nvec = CP * E // 16
    ecols = E // 16
    tbufs = (tb0, tb1)
    ssems = (ss0, ss1)
    store_d = [None, None]
    u = 0
    for chunk in range(spw // CP):
        s0 = base + chunk * CP
        pltpu.sync_copy(pos.at[pl.ds(s0, CP)], pos_buf)
        for b in range(B):
            cur = u % 2
            if store_d[cur] is not None:
                store_d[cur].wait()
            tb = tbufs[cur]
            pltpu.sync_copy(tok.at[b, pl.ds(s0, CP)], tb)

            def body(i, carry):
                r = i // ecols
                sl = pl.ds((i % ecols) * 16, 16)
                plsc.addupdate(tb.at[r, sl], pos_buf[r, sl])
                return carry

            lax.fori_loop(0, nvec, body, 0, unroll=8)
            store_d[cur] = pltpu.async_copy(
                tb, out.at[b, pl.ds(s0, CP)], ssems[cur])
            u += 1
    store_d[0].wait()
    store_d[1].wait()


def kernel(token_embedding, pos_table):
    B, S, E = token_embedding.shape
    f = pl.kernel(
        _sc_body,
        out_type=jax.ShapeDtypeStruct((B, S, E), token_embedding.dtype),
        mesh=plsc.VectorSubcoreMesh(core_axis_name="c", subcore_axis_name="s"),
        scratch_types=[
            pltpu.VMEM((CP, E), jnp.float32),
            pltpu.VMEM((CP, E), jnp.float32),
            pltpu.VMEM((CP, E), jnp.float32),
            pltpu.SemaphoreType.DMA,
            pltpu.SemaphoreType.DMA,
        ],
    )
    return f(token_embedding, pos_table)


# speedup vs baseline: 3.2288x; 2.4983x over previous
import jax
import jax.numpy as jnp
from jax import lax
from jax.experimental import pallas as pl
from jax.experimental.pallas import tpu as pltpu
from jax.experimental.pallas import tpu_sc as plsc

NC, NS = 2, 16
NW = NC * NS
CP = 16
LANES = 16
NBUF = 4
PREF = 2


def _sc_body(tok, pos, out, pb0, pb1, tb0, tb1, tb2, tb3,
             ps0, ps1, ls0, ls1, ls2, ls3, ss0, ss1, ss2, ss3):
    B, S, E = tok.shape
    spw = S // NW
    wid = lax.axis_index("s") * NC + lax.axis_index("c")
    base = wid * spw

    nchunks = spw // CP
    nunits = nchunks * B
    nvec = CP * E // LANES
    ecols = E // LANES

    tbufs = (tb0, tb1, tb2, tb3)
    lsems = (ls0, ls1, ls2, ls3)
    ssems = (ss0, ss1, ss2, ss3)
    pbufs = (pb0, pb1)
    psems = (ps0, ps1)

    load_d = [None] * NBUF
    store_d = [None] * NBUF
    pos_d = [None, None]

    def unit_pos(u):
        chunk, b = u // B, u % B
        return chunk, b, base + chunk * CP

    pos_d[0] = pltpu.async_copy(pos.at[pl.ds(base, CP)], pbufs[0], psems[0])
    for up in range(min(PREF, nunits)):
        _, b, s0 = unit_pos(up)
        load_d[up % NBUF] = pltpu.async_copy(
            tok.at[b, pl.ds(s0, CP)], tbufs[up % NBUF], lsems[up % NBUF])

    for u in range(nunits):
        chunk, b, s0 = unit_pos(u)
        if b == 0:
            pos_d[chunk % 2].wait()
            pos_d[chunk % 2] = None
            if chunk + 1 < nchunks:
                nxt = (chunk + 1) % 2
                pos_d[nxt] = pltpu.async_copy(
                    pos.at[pl.ds(base + (chunk + 1) * CP, CP)],
                    pbufs[nxt], psems[nxt])
        up = u + PREF
        if up < nunits:
            slot = up % NBUF
            if store_d[slot] is not None:
                store_d[slot].wait()
                store_d[slot] = None
            _, ub, us0 = unit_pos(up)
            load_d[slot] = pltpu.async_copy(
                tok.at[ub, pl.ds(us0, CP)], tbufs[slot], lsems[slot])

        cur = u % NBUF
        load_d[cur].wait()
        load_d[cur] = None
        tb = tbufs[cur]
        pb = pbufs[chunk % 2]

        @plsc.parallel_loop(0, nvec, 1, unroll=8)
        def body(i):
            r = i // ecols
            sl = pl.ds((i % ecols) * LANES, LANES)
            plsc.addupdate(tb.at[r, sl], pb[r, sl])

        store_d[cur] = pltpu.async_copy(
            tb, out.at[b, pl.ds(s0, CP)], ssems[cur])

    for slot in range(NBUF):
        if store_d[slot] is not None:
            store_d[slot].wait()


def kernel(token_embedding, pos_table):
    B, S, E = token_embedding.shape
    f = pl.kernel(
        _sc_body,
        out_type=jax.ShapeDtypeStruct((B, S, E), token_embedding.dtype),
        mesh=plsc.VectorSubcoreMesh(core_axis_name="c", subcore_axis_name="s"),
        scratch_types=[
            pltpu.VMEM((CP, E), jnp.float32),
            pltpu.VMEM((CP, E), jnp.float32),
            pltpu.VMEM((CP, E), jnp.float32),
            pltpu.VMEM((CP, E), jnp.float32),
            pltpu.VMEM((CP, E), jnp.float32),
            pltpu.VMEM((CP, E), jnp.float32),
            pltpu.SemaphoreType.DMA,
            pltpu.SemaphoreType.DMA,
            pltpu.SemaphoreType.DMA,
            pltpu.SemaphoreType.DMA,
            pltpu.SemaphoreType.DMA,
            pltpu.SemaphoreType.DMA,
            pltpu.SemaphoreType.DMA,
            pltpu.SemaphoreType.DMA,
            pltpu.SemaphoreType.DMA,
            pltpu.SemaphoreType.DMA,
        ],
    )
    return f(token_embedding, pos_table)
